# Initial kernel scaffold; baseline (speedup 1.0000x reference)
import jax
import jax.numpy as jnp
from jax.experimental import pallas as pl


def kernel(x, token_table, pos_table):
    raise NotImplementedError("write your pallas kernel here")


# speedup vs baseline: 2.6527x; 2.6527x over previous
import functools

import jax
import jax.numpy as jnp
from jax import lax
from jax.experimental import pallas as pl
from jax.experimental.pallas import tpu as pltpu
from jax.experimental.pallas import tpu_sc as plsc

D = 128
B = 4096
L = 50
ROWS = B * L
NW = 32
ROWS_PER_W = ROWS // NW
CHUNK_ROWS = 200
GPC = 2
G = CHUNK_ROWS // GPC
NCHUNK = ROWS_PER_W // CHUNK_ROWS
NCHUNK_TOTAL = ROWS // CHUNK_ROWS
NVEC = D // 16


def _emb_body(x_hbm, tok_hbm, pos_hbm, out_hbm, idx_v, rows_v, pos_v, gsem, osem):
    wid = lax.axis_index("s") * 2 + lax.axis_index("c")
    pltpu.sync_copy(pos_hbm, pos_v)

    def chunk_body(c, carry):
        crow = wid * NCHUNK + c
        pltpu.sync_copy(x_hbm.at[crow], idx_v)
        cps = [
            pltpu.async_copy(
                tok_hbm.at[idx_v.at[j]], rows_v.at[pl.ds(j * G, G)], gsem
            )
            for j in range(GPC)
        ]
        for cp in cps:
            cp.wait()

        def l_body(l, inner_carry):
            pv = [pos_v[l, pl.ds(jj * 16, 16)] for jj in range(NVEC)]
            for sblk in range(CHUNK_ROWS // L):
                row = sblk * L + l
                for jj in range(NVEC):
                    sl = pl.ds(jj * 16, 16)
                    rows_v[row, sl] = rows_v[row, sl] + pv[jj]
            return inner_carry

        lax.fori_loop(0, L, l_body, 0)
        pltpu.async_copy(
            rows_v, out_hbm.at[pl.ds(crow * CHUNK_ROWS, CHUNK_ROWS)], osem
        ).wait()
        return carry

    lax.fori_loop(0, NCHUNK, chunk_body, 0)


_emb = functools.partial(
    pl.kernel,
    out_type=jax.ShapeDtypeStruct((ROWS, D), jnp.float32),
    mesh=plsc.VectorSubcoreMesh(core_axis_name="c", subcore_axis_name="s"),
    scratch_types=[
        pltpu.VMEM((GPC, G), jnp.int32),
        pltpu.VMEM((CHUNK_ROWS, D), jnp.float32),
        pltpu.VMEM((L, D), jnp.float32),
        pltpu.SemaphoreType.DMA,
        pltpu.SemaphoreType.DMA,
    ],
)(_emb_body)


def kernel(x, token_table, pos_table):
    x3 = x.reshape(NCHUNK_TOTAL, GPC, G).astype(jnp.int32)
    out = _emb(x3, token_table, pos_table)
    return out.reshape(B, L, D)


# speedup vs baseline: 3.2799x; 1.2365x over previous
import functools

import jax
import jax.numpy as jnp
from jax import lax
from jax.experimental import pallas as pl
from jax.experimental.pallas import tpu as pltpu
from jax.experimental.pallas import tpu_sc as plsc

D = 128
B = 4096
L = 50
ROWS = B * L
NW = 32
ROWS_PER_W = ROWS // NW
CHUNK_ROWS = 200
GPC = 2
G = CHUNK_ROWS // GPC
NCHUNK = ROWS_PER_W // CHUNK_ROWS
NCHUNK_TOTAL = ROWS // CHUNK_ROWS
NVEC = D // 16
NBUF = 4


def _issue_chunk(x_hbm, tok_hbm, idx_v, rows_v, gsem, kb, crow):
    pltpu.sync_copy(x_hbm.at[crow], idx_v.at[kb])
    for j in range(GPC):
        pltpu.async_copy(
            tok_hbm.at[idx_v.at[kb].at[j]],
            rows_v.at[kb].at[pl.ds(j * G, G)],
            gsem[kb],
        )


def _wait_gather(tok_hbm, idx_v, rows_v, gsem, kb):
    for j in range(GPC):
        pltpu.make_async_copy(
            tok_hbm.at[idx_v.at[kb].at[j]],
            rows_v.at[kb].at[pl.ds(j * G, G)],
            gsem[kb],
        ).wait()


def _wait_ocopy(rows_v, out_hbm, osem, kb):
    pltpu.make_async_copy(
        rows_v.at[kb], out_hbm.at[pl.ds(0, CHUNK_ROWS)], osem[kb]
    ).wait()


def _add_pos(rows_v, pos_v, kb):
    def l_body(l, carry):
        pv = [pos_v[l, pl.ds(jj * 16, 16)] for jj in range(NVEC)]
        for sblk in range(CHUNK_ROWS // L):
            row = sblk * L + l
            for jj in range(NVEC):
                sl = pl.ds(jj * 16, 16)
                rows_v[kb, row, sl] = rows_v[kb, row, sl] + pv[jj]
        return carry

    lax.fori_loop(0, L, l_body, 0)


def _emb_body(x_hbm, tok_hbm, pos_hbm, out_hbm, idx_v, rows_v, pos_v,
              gs0, gs1, gs2, gs3, os0, os1, os2, os3):
    gsem = [gs0, gs1, gs2, gs3]
    osem = [os0, os1, os2, os3]
    wid = lax.axis_index("s") * 2 + lax.axis_index("c")
    base = wid * NCHUNK
    pltpu.sync_copy(pos_hbm, pos_v)

    for c in range(NBUF - 1):
        _issue_chunk(x_hbm, tok_hbm, idx_v, rows_v, gsem, c, base + c)

    def group_body(g, carry):
        for k in range(NBUF):
            c = g * NBUF + k
            crow = base + c
            _wait_gather(tok_hbm, idx_v, rows_v, gsem, k)
            _add_pos(rows_v, pos_v, k)
            pltpu.async_copy(
                rows_v.at[k],
                out_hbm.at[pl.ds(crow * CHUNK_ROWS, CHUNK_ROWS)],
                osem[k],
            )
            kn = (k + NBUF - 1) % NBUF
            cn = c + NBUF - 1

            @pl.when(cn < NCHUNK)
            def _issue_next():
                @pl.when(c >= 1)
                def _drain_prev():
                    _wait_ocopy(rows_v, out_hbm, osem, kn)

                _issue_chunk(x_hbm, tok_hbm, idx_v, rows_v, gsem, kn,
                             base + cn)

        return carry

    lax.fori_loop(0, NCHUNK // NBUF, group_body, 0)
    for k in range(NBUF):
        _wait_ocopy(rows_v, out_hbm, osem, k)


_emb = functools.partial(
    pl.kernel,
    out_type=jax.ShapeDtypeStruct((ROWS, D), jnp.float32),
    mesh=plsc.VectorSubcoreMesh(core_axis_name="c", subcore_axis_name="s"),
    scratch_types=[
        pltpu.VMEM((NBUF, GPC, G), jnp.int32),
        pltpu.VMEM((NBUF, CHUNK_ROWS, D), jnp.float32),
        pltpu.VMEM((L, D), jnp.float32),
    ] + [pltpu.SemaphoreType.DMA] * (2 * NBUF),
)(_emb_body)


def kernel(x, token_table, pos_table):
    x3 = x.reshape(NCHUNK_TOTAL, GPC, G).astype(jnp.int32)
    out = _emb(x3, token_table, pos_table)
    return out.reshape(B, L, D)


# speedup vs baseline: 3.2888x; 1.0027x over previous
import functools

import jax
import jax.numpy as jnp
from jax import lax
from jax.experimental import pallas as pl
from jax.experimental.pallas import tpu as pltpu
from jax.experimental.pallas import tpu_sc as plsc

D = 128
B = 4096
L = 50
ROWS = B * L
NW = 32
ROWS_PER_W = ROWS // NW
CHUNK_ROWS = 200
GPC = 2
G = CHUNK_ROWS // GPC
NCHUNK = ROWS_PER_W // CHUNK_ROWS
NCHUNK_TOTAL = ROWS // CHUNK_ROWS
NVEC = D // 16
NBUF = 4


def _issue_chunk(x_hbm, tok_hbm, idx_v, rows_v, gsem, kb, crow):
    pltpu.sync_copy(x_hbm.at[crow], idx_v.at[kb])
    for j in range(GPC):
        pltpu.async_copy(
            tok_hbm.at[idx_v.at[kb].at[j]],
            rows_v.at[kb].at[pl.ds(j * G, G)],
            gsem[kb],
        )


def _wait_gather(tok_hbm, idx_v, rows_v, gsem, kb):
    for j in range(GPC):
        pltpu.make_async_copy(
            tok_hbm.at[idx_v.at[kb].at[j]],
            rows_v.at[kb].at[pl.ds(j * G, G)],
            gsem[kb],
        ).wait()


def _wait_ocopy(rows_v, out_hbm, osem, kb):
    pltpu.make_async_copy(
        rows_v.at[kb], out_hbm.at[pl.ds(0, CHUNK_ROWS)], osem[kb]
    ).wait()


def _add_pos(rows_v, pos_v, kb):
    @plsc.parallel_loop(0, L, unroll=2)
    def l_body(l):
        pv = [pos_v[l, pl.ds(jj * 16, 16)] for jj in range(NVEC)]
        for sblk in range(CHUNK_ROWS // L):
            row = sblk * L + l
            for jj in range(NVEC):
                sl = pl.ds(jj * 16, 16)
                rows_v[kb, row, sl] = rows_v[kb, row, sl] + pv[jj]


def _emb_body(x_hbm, tok_hbm, pos_hbm, out_hbm, idx_v, rows_v, pos_v,
              gs0, gs1, gs2, gs3, os0, os1, os2, os3):
    gsem = [gs0, gs1, gs2, gs3]
    osem = [os0, os1, os2, os3]
    wid = lax.axis_index("s") * 2 + lax.axis_index("c")
    base = wid * NCHUNK
    pltpu.sync_copy(pos_hbm, pos_v)

    for c in range(NBUF - 1):
        _issue_chunk(x_hbm, tok_hbm, idx_v, rows_v, gsem, c, base + c)

    def group_body(g, carry):
        for k in range(NBUF):
            c = g * NBUF + k
            crow = base + c
            _wait_gather(tok_hbm, idx_v, rows_v, gsem, k)
            _add_pos(rows_v, pos_v, k)
            pltpu.async_copy(
                rows_v.at[k],
                out_hbm.at[pl.ds(crow * CHUNK_ROWS, CHUNK_ROWS)],
                osem[k],
            )
            kn = (k + NBUF - 1) % NBUF
            cn = c + NBUF - 1

            @pl.when(cn < NCHUNK)
            def _issue_next():
                @pl.when(c >= 1)
                def _drain_prev():
                    _wait_ocopy(rows_v, out_hbm, osem, kn)

                _issue_chunk(x_hbm, tok_hbm, idx_v, rows_v, gsem, kn,
                             base + cn)

        return carry

    lax.fori_loop(0, NCHUNK // NBUF, group_body, 0)
    for k in range(NBUF):
        _wait_ocopy(rows_v, out_hbm, osem, k)


_emb = functools.partial(
    pl.kernel,
    out_type=jax.ShapeDtypeStruct((ROWS, D), jnp.float32),
    mesh=plsc.VectorSubcoreMesh(core_axis_name="c", subcore_axis_name="s"),
    scratch_types=[
        pltpu.VMEM((NBUF, GPC, G), jnp.int32),
        pltpu.VMEM((NBUF, CHUNK_ROWS, D), jnp.float32),
        pltpu.VMEM((L, D), jnp.float32),
    ] + [pltpu.SemaphoreType.DMA] * (2 * NBUF),
)(_emb_body)


def kernel(x, token_table, pos_table):
    x3 = x.reshape(NCHUNK_TOTAL, GPC, G).astype(jnp.int32)
    out = _emb(x3, token_table, pos_table)
    return out.reshape(B, L, D)


# speedup vs baseline: 4.9369x; 1.5011x over previous
import functools

import jax
import jax.numpy as jnp
from jax import lax
from jax.experimental import pallas as pl
from jax.experimental.pallas import tpu as pltpu
from jax.experimental.pallas import tpu_sc as plsc

D = 128
B = 4096
L = 50
LPAD = 56
NW = 32
SEQ_PER_W = B // NW
CHUNK_SEQ = 4
NCHUNK = SEQ_PER_W // CHUNK_SEQ
NVEC = D // 16
NBUF = 4


def _issue_chunk(x_hbm, tok_hbm, idx_v, rows_v, gsem, kb, b0):
    pltpu.sync_copy(x_hbm.at[pl.ds(b0, CHUNK_SEQ)], idx_v.at[kb])
    for s in range(CHUNK_SEQ):
        pltpu.async_copy(
            tok_hbm.at[idx_v.at[kb].at[s]],
            rows_v.at[kb].at[s].at[pl.ds(0, L)],
            gsem[kb],
        )


def _wait_gather(tok_hbm, idx_v, rows_v, gsem, kb):
    for s in range(CHUNK_SEQ):
        pltpu.make_async_copy(
            tok_hbm.at[idx_v.at[kb].at[s]],
            rows_v.at[kb].at[s].at[pl.ds(0, L)],
            gsem[kb],
        ).wait()


def _wait_ocopy(rows_v, out_hbm, osem, kb):
    pltpu.make_async_copy(
        rows_v.at[kb], out_hbm.at[pl.ds(0, CHUNK_SEQ)], osem[kb]
    ).wait()


def _add_pos(rows_v, pos_v, kb):
    @plsc.parallel_loop(0, L, unroll=2)
    def l_body(l):
        pv = [pos_v[l, pl.ds(jj * 16, 16)] for jj in range(NVEC)]
        for s in range(CHUNK_SEQ):
            for jj in range(NVEC):
                sl = pl.ds(jj * 16, 16)
                rows_v[kb, s, l, sl] = rows_v[kb, s, l, sl] + pv[jj]


def _emb_body(x_hbm, tok_hbm, pos_hbm, out_hbm, idx_v, rows_v, pos_v,
              gs0, gs1, gs2, gs3, os0, os1, os2, os3):
    gsem = [gs0, gs1, gs2, gs3]
    osem = [os0, os1, os2, os3]
    wid = lax.axis_index("s") * 2 + lax.axis_index("c")
    base = wid * NCHUNK
    pltpu.sync_copy(pos_hbm, pos_v)

    for c in range(NBUF - 1):
        _issue_chunk(x_hbm, tok_hbm, idx_v, rows_v, gsem, c,
                     (base + c) * CHUNK_SEQ)

    def group_body(g, carry):
        for k in range(NBUF):
            c = g * NBUF + k
            b0 = (base + c) * CHUNK_SEQ
            _wait_gather(tok_hbm, idx_v, rows_v, gsem, k)
            _add_pos(rows_v, pos_v, k)
            pltpu.async_copy(
                rows_v.at[k], out_hbm.at[pl.ds(b0, CHUNK_SEQ)], osem[k]
            )
            kn = (k + NBUF - 1) % NBUF
            cn = c + NBUF - 1

            @pl.when(cn < NCHUNK)
            def _issue_next():
                @pl.when(c >= 1)
                def _drain_prev():
                    _wait_ocopy(rows_v, out_hbm, osem, kn)

                _issue_chunk(x_hbm, tok_hbm, idx_v, rows_v, gsem, kn,
                             (base + cn) * CHUNK_SEQ)

        return carry

    lax.fori_loop(0, NCHUNK // NBUF, group_body, 0)
    for k in range(NBUF):
        _wait_ocopy(rows_v, out_hbm, osem, k)


_emb = functools.partial(
    pl.kernel,
    out_type=jax.ShapeDtypeStruct((B, LPAD, D), jnp.float32),
    mesh=plsc.VectorSubcoreMesh(core_axis_name="c", subcore_axis_name="s"),
    scratch_types=[
        pltpu.VMEM((NBUF, CHUNK_SEQ, L), jnp.int32),
        pltpu.VMEM((NBUF, CHUNK_SEQ, LPAD, D), jnp.float32),
        pltpu.VMEM((L, D), jnp.float32),
    ] + [pltpu.SemaphoreType.DMA] * (2 * NBUF),
)(_emb_body)


def kernel(x, token_table, pos_table):
    out = _emb(x.astype(jnp.int32), token_table, pos_table)
    return out[:, :L, :]


# speedup vs baseline: 5.7631x; 1.1674x over previous
import functools

import jax
import jax.numpy as jnp
from jax import lax
from jax.experimental import pallas as pl
from jax.experimental.pallas import tpu as pltpu
from jax.experimental.pallas import tpu_sc as plsc

D = 128
B = 4096
L = 50
LPAD = 56
NW = 32
SEQ_PER_W = B // NW
CHUNK_SEQ = 4
NCHUNK = SEQ_PER_W // CHUNK_SEQ
NVEC = D // 16
NBUF = 4


def _issue_chunk(x_hbm, tok_hbm, idx_v, rows_v, gsem, kb, b0):
    pltpu.sync_copy(x_hbm.at[pl.ds(b0, CHUNK_SEQ)], idx_v.at[kb])
    for s in range(CHUNK_SEQ):
        pltpu.async_copy(
            tok_hbm.at[idx_v.at[kb].at[s]],
            rows_v.at[kb].at[s],
            gsem[kb],
        )


def _wait_gather(tok_hbm, idx_v, rows_v, gsem, kb):
    for s in range(CHUNK_SEQ):
        pltpu.make_async_copy(
            tok_hbm.at[idx_v.at[kb].at[s]],
            rows_v.at[kb].at[s],
            gsem[kb],
        ).wait()


def _wait_ocopy(rows_v, out_hbm, osem, kb):
    pltpu.make_async_copy(
        rows_v.at[kb], out_hbm.at[pl.ds(0, CHUNK_SEQ)], osem[kb]
    ).wait()


def _add_pos(rows_v, pos_v, kb):
    @plsc.parallel_loop(0, L, unroll=2)
    def l_body(l):
        pv = [pos_v[l, pl.ds(jj * 16, 16)] for jj in range(NVEC)]
        for s in range(CHUNK_SEQ):
            for jj in range(NVEC):
                sl = pl.ds(jj * 16, 16)
                rows_v[kb, s, l, sl] = rows_v[kb, s, l, sl] + pv[jj]


def _emb_body(x_hbm, tok_hbm, pos_hbm, out_hbm, idx_v, rows_v, pos_v,
              gs0, gs1, gs2, gs3, os0, os1, os2, os3):
    gsem = [gs0, gs1, gs2, gs3]
    osem = [os0, os1, os2, os3]
    wid = lax.axis_index("s") * 2 + lax.axis_index("c")
    base = wid * NCHUNK
    pltpu.sync_copy(pos_hbm, pos_v)

    for c in range(NBUF - 1):
        _issue_chunk(x_hbm, tok_hbm, idx_v, rows_v, gsem, c,
                     (base + c) * CHUNK_SEQ)

    def group_body(g, carry):
        for k in range(NBUF):
            c = g * NBUF + k
            b0 = (base + c) * CHUNK_SEQ
            _wait_gather(tok_hbm, idx_v, rows_v, gsem, k)
            _add_pos(rows_v, pos_v, k)
            pltpu.async_copy(
                rows_v.at[k], out_hbm.at[pl.ds(b0, CHUNK_SEQ)], osem[k]
            )
            kn = (k + NBUF - 1) % NBUF
            cn = c + NBUF - 1

            @pl.when(cn < NCHUNK)
            def _issue_next():
                @pl.when(c >= 1)
                def _drain_prev():
                    _wait_ocopy(rows_v, out_hbm, osem, kn)

                _issue_chunk(x_hbm, tok_hbm, idx_v, rows_v, gsem, kn,
                             (base + cn) * CHUNK_SEQ)

        return carry

    lax.fori_loop(0, NCHUNK // NBUF, group_body, 0)
    for k in range(NBUF):
        _wait_ocopy(rows_v, out_hbm, osem, k)


_emb = functools.partial(
    pl.kernel,
    out_type=jax.ShapeDtypeStruct((B, L, D), jnp.float32),
    mesh=plsc.VectorSubcoreMesh(core_axis_name="c", subcore_axis_name="s"),
    scratch_types=[
        pltpu.VMEM((NBUF, CHUNK_SEQ, L), jnp.int32),
        pltpu.VMEM((NBUF, CHUNK_SEQ, L, D), jnp.float32),
        pltpu.VMEM((L, D), jnp.float32),
    ] + [pltpu.SemaphoreType.DMA] * (2 * NBUF),
)(_emb_body)


def kernel(x, token_table, pos_table):
    return _emb(x.astype(jnp.int32), token_table, pos_table)
